# baseline (speedup 1.0000x reference)
import functools

import jax
import jax.numpy as jnp
from jax import lax
from jax.experimental import pallas as pl
from jax.experimental.pallas import tpu as pltpu
from jax.experimental.pallas import tpu_sc as plsc

MEM_LENGTH = 8192
HIDDEN = 768
BATCH = 4

_NC = 2
_NS = 16
_L = 16
_NW = _NC * _NS
_LPW = MEM_LENGTH // _NW
_C = 8
_NCHUNK = _LPW // _C
_NJ = HIDDEN // _L
_JB = 8

_GATHER_DNUMS = lax.GatherDimensionNumbers(
    offset_dims=(), collapsed_slice_dims=(0,), start_index_map=(0,))


def _lane_shuffle(v, perm):
    return lax.gather(v, perm[:, None], _GATHER_DNUMS, slice_sizes=(1,),
                      mode=lax.GatherScatterMode.PROMISE_IN_BOUNDS)


def _allsum_vec(v):
    idx = lax.iota(jnp.int32, 16)
    for sh in (8, 4, 2, 1):
        perm = lax.bitwise_xor(idx, jnp.int32(sh))
        v = v + _lane_shuffle(v, perm)
    return v


def _rsqrt_vec(x):
    i = lax.bitcast_convert_type(x, jnp.int32)
    i = jnp.int32(0x5F3759DF) - lax.shift_right_logical(i, 1)
    y = lax.bitcast_convert_type(i, jnp.float32)
    for _ in range(3):
        y = y * (1.5 - 0.5 * x * y * y)
    return y


def _sc_body(mem, emb, gamma, beta, out, gv, bv,
             e0, e1,
             x00, x01, x02, x03, x10, x11, x12, x13,
             o00, o01, o02, o03, o10, o11, o12, o13,
             rs_v, ms_v, sin0, sin1, sout0, sout1):
    cid = lax.axis_index("c")
    sid = lax.axis_index("s")
    wid = sid * _NC + cid
    base = wid * _LPW
    pltpu.sync_copy(gamma, gv)
    pltpu.sync_copy(beta, bv)

    xs = ((x00, x01, x02, x03), (x10, x11, x12, x13))
    os_ = ((o00, o01, o02, o03), (o10, o11, o12, o13))
    es = (e0, e1)
    sins = (sin0, sin1)
    souts = (sout0, sout1)
    zero = jnp.zeros((_L,), jnp.float32)

    def issue_ins(slot, t):
        l0 = base + t * _C
        pltpu.async_copy(emb.at[pl.ds(l0, _C)], es[slot], sins[slot])
        for b in range(BATCH):
            pltpu.async_copy(mem.at[b, pl.ds(l0, _C)], xs[slot][b], sins[slot])

    def wait_ins(slot, t):
        l0 = base + t * _C
        pltpu.make_async_copy(emb.at[pl.ds(l0, _C)], es[slot], sins[slot]).wait()
        for b in range(BATCH):
            pltpu.make_async_copy(
                mem.at[b, pl.ds(l0, _C)], xs[slot][b], sins[slot]).wait()

    def start_outs(slot, t):
        l0 = base + t * _C
        for b in range(BATCH):
            pltpu.async_copy(os_[slot][b], out.at[b, pl.ds(l0, _C)], souts[slot])

    def drain_outs(slot, t):
        l0 = base + t * _C
        for b in range(BATCH):
            pltpu.make_async_copy(
                os_[slot][b], out.at[b, pl.ds(l0, _C)], souts[slot]).wait()

    def compute(slot):
        xb = xs[slot]
        ob = os_[slot]
        ev = es[slot]

        def pass1(r, rc):
            s = [[zero, zero] for _ in range(BATCH)]
            sq = [[zero, zero] for _ in range(BATCH)]
            for j in range(_NJ):
                sl = pl.ds(j * _L, _L)
                e = ev[r, sl]
                p = j & 1
                for b in range(BATCH):
                    v = xb[b][r, sl] + e
                    xb[b][r, sl] = v
                    s[b][p] = s[b][p] + v
                    sq[b][p] = sq[b][p] + v * v
            for b in range(BATCH):
                tot = _allsum_vec(s[b][0] + s[b][1])
                totsq = _allsum_vec(sq[b][0] + sq[b][1])
                mean = tot * (1.0 / HIDDEN)
                var = totsq * (1.0 / HIDDEN) - mean * mean
                rstd = _rsqrt_vec(var + 1e-5)
                rs_v[b, r, :] = rstd
                ms_v[b, r, :] = mean * rstd
            return rc

        lax.fori_loop(0, _C, pass1, 0)

        for jb in range(_NJ // _JB):
            gs = [gv[pl.ds((jb * _JB + k) * _L, _L)] for k in range(_JB)]
            bs = [bv[pl.ds((jb * _JB + k) * _L, _L)] for k in range(_JB)]
            for b in range(BATCH):
                def pass2(r, rc, jb=jb, b=b, gs=gs, bs=bs):
                    rs = rs_v[b, r, :]
                    ms = ms_v[b, r, :]
                    for k in range(_JB):
                        sl = pl.ds((jb * _JB + k) * _L, _L)
                        v = xb[b][r, sl]
                        ob[b][r, sl] = (v * rs - ms) * gs[k] + bs[k]
                    return rc

                lax.fori_loop(0, _C, pass2, 0)

    issue_ins(0, 0)
    issue_ins(1, 1)

    def outer(g, carry):
        for slot in range(2):
            t = g * 2 + slot
            wait_ins(slot, t)

            @pl.when(t >= 2)
            def _():
                drain_outs(slot, t - 2)

            compute(slot)
            start_outs(slot, t)

            @pl.when(t < _NCHUNK - 2)
            def _():
                issue_ins(slot, t + 2)

        return carry

    lax.fori_loop(0, _NCHUNK // 2, outer, 0)
    drain_outs(0, _NCHUNK - 2)
    drain_outs(1, _NCHUNK - 1)


@jax.jit
def kernel(mem, emb_table, gamma, beta):
    mesh = plsc.VectorSubcoreMesh(core_axis_name="c", subcore_axis_name="s")
    chunk_t = pltpu.VMEM((_C, HIDDEN), jnp.float32)
    run = pl.kernel(
        _sc_body,
        mesh=mesh,
        out_type=jax.ShapeDtypeStruct((BATCH, MEM_LENGTH, HIDDEN), jnp.float32),
        scratch_types=(
            [pltpu.VMEM((HIDDEN,), jnp.float32)] * 2
            + [chunk_t] * 2
            + [chunk_t] * 8
            + [chunk_t] * 8
            + [pltpu.VMEM((BATCH, _C, _L), jnp.float32)] * 2
            + [pltpu.SemaphoreType.DMA] * 4
        ),
    )
    return run(mem, emb_table, gamma, beta)


# speedup vs baseline: 1.4423x; 1.4423x over previous
import functools

import jax
import jax.numpy as jnp
from jax import lax
from jax.experimental import pallas as pl
from jax.experimental.pallas import tpu as pltpu
from jax.experimental.pallas import tpu_sc as plsc

MEM_LENGTH = 8192
HIDDEN = 768
BATCH = 4

_TCB = 3
_BL = 512

_NC = 2
_NS = 16
_L = 16
_NW = _NC * _NS
_LPW = MEM_LENGTH // _NW
_C = 8
_NCHUNK = _LPW // _C
_NJ = HIDDEN // _L
_JB = 8

_GATHER_DNUMS = lax.GatherDimensionNumbers(
    offset_dims=(), collapsed_slice_dims=(0,), start_index_map=(0,))


def _lane_shuffle(v, perm):
    return lax.gather(v, perm[:, None], _GATHER_DNUMS, slice_sizes=(1,),
                      mode=lax.GatherScatterMode.PROMISE_IN_BOUNDS)


def _allsum_vec(v):
    idx = lax.iota(jnp.int32, 16)
    for sh in (8, 4, 2, 1):
        perm = lax.bitwise_xor(idx, jnp.int32(sh))
        v = v + _lane_shuffle(v, perm)
    return v


def _rsqrt_vec(x):
    i = lax.bitcast_convert_type(x, jnp.int32)
    i = jnp.int32(0x5F3759DF) - lax.shift_right_logical(i, 1)
    y = lax.bitcast_convert_type(i, jnp.float32)
    for _ in range(3):
        y = y * (1.5 - 0.5 * x * y * y)
    return y


def _ln_body(mem_ref, emb_ref, gamma_ref, beta_ref, out_ref):
    x = mem_ref[0] + emb_ref[...]
    mean = jnp.mean(x, axis=-1, keepdims=True)
    xc = x - mean
    var = jnp.mean(xc * xc, axis=-1, keepdims=True)
    inv = jax.lax.rsqrt(var + 1e-5)
    out_ref[0] = xc * inv * gamma_ref[...] + beta_ref[...]


def _tc_part(mem, emb_table, gamma, beta, nb, out_nb):
    grid = (nb, MEM_LENGTH // _BL)
    return pl.pallas_call(
        _ln_body,
        grid=grid,
        in_specs=[
            pl.BlockSpec((1, _BL, HIDDEN), lambda b, i: (b, i, 0)),
            pl.BlockSpec((_BL, HIDDEN), lambda b, i: (i, 0)),
            pl.BlockSpec((HIDDEN,), lambda b, i: (0,)),
            pl.BlockSpec((HIDDEN,), lambda b, i: (0,)),
        ],
        out_specs=pl.BlockSpec((1, _BL, HIDDEN), lambda b, i: (b, i, 0)),
        out_shape=jax.ShapeDtypeStruct((out_nb, MEM_LENGTH, HIDDEN), jnp.float32),
    )(mem, emb_table, gamma, beta)


def _sc_body(nb, b_lo, mem, emb, gamma, beta, out, *scr):
    cid = lax.axis_index("c")
    sid = lax.axis_index("s")
    wid = sid * _NC + cid
    base = wid * _LPW

    gv, bv = scr[0], scr[1]
    es = scr[2:4]
    xs = (scr[4:4 + nb], scr[4 + nb:4 + 2 * nb])
    o0 = 4 + 2 * nb
    os_ = (scr[o0:o0 + nb], scr[o0 + nb:o0 + 2 * nb])
    rs_v, ms_v = scr[o0 + 2 * nb], scr[o0 + 2 * nb + 1]
    sins = scr[o0 + 2 * nb + 2:o0 + 2 * nb + 4]
    souts = scr[o0 + 2 * nb + 4:o0 + 2 * nb + 6]

    pltpu.sync_copy(gamma, gv)
    pltpu.sync_copy(beta, bv)
    zero = jnp.zeros((_L,), jnp.float32)

    def issue_ins(slot, t):
        l0 = base + t * _C
        pltpu.async_copy(emb.at[pl.ds(l0, _C)], es[slot], sins[slot])
        for b in range(nb):
            pltpu.async_copy(mem.at[b_lo + b, pl.ds(l0, _C)], xs[slot][b],
                             sins[slot])

    def wait_ins(slot, t):
        l0 = base + t * _C
        pltpu.make_async_copy(emb.at[pl.ds(l0, _C)], es[slot], sins[slot]).wait()
        for b in range(nb):
            pltpu.make_async_copy(
                mem.at[b_lo + b, pl.ds(l0, _C)], xs[slot][b], sins[slot]).wait()

    def start_outs(slot, t):
        l0 = base + t * _C
        for b in range(nb):
            pltpu.async_copy(os_[slot][b], out.at[b, pl.ds(l0, _C)], souts[slot])

    def drain_outs(slot, t):
        l0 = base + t * _C
        for b in range(nb):
            pltpu.make_async_copy(
                os_[slot][b], out.at[b, pl.ds(l0, _C)], souts[slot]).wait()

    def compute(slot):
        xb = xs[slot]
        ob = os_[slot]
        ev = es[slot]

        def pass1(r, rc):
            s = [[zero, zero] for _ in range(nb)]
            sq = [[zero, zero] for _ in range(nb)]
            for j in range(_NJ):
                sl = pl.ds(j * _L, _L)
                e = ev[r, sl]
                p = j & 1
                for b in range(nb):
                    v = xb[b][r, sl] + e
                    xb[b][r, sl] = v
                    s[b][p] = s[b][p] + v
                    sq[b][p] = sq[b][p] + v * v
            for b in range(nb):
                tot = _allsum_vec(s[b][0] + s[b][1])
                totsq = _allsum_vec(sq[b][0] + sq[b][1])
                mean = tot * (1.0 / HIDDEN)
                var = totsq * (1.0 / HIDDEN) - mean * mean
                rstd = _rsqrt_vec(var + 1e-5)
                rs_v[b, r, :] = rstd
                ms_v[b, r, :] = mean * rstd
            return rc

        lax.fori_loop(0, _C, pass1, 0)

        for jb in range(_NJ // _JB):
            gs = [gv[pl.ds((jb * _JB + k) * _L, _L)] for k in range(_JB)]
            bs = [bv[pl.ds((jb * _JB + k) * _L, _L)] for k in range(_JB)]
            for b in range(nb):
                def pass2(r, rc, jb=jb, b=b, gs=gs, bs=bs):
                    rs = rs_v[b, r, :]
                    ms = ms_v[b, r, :]
                    for k in range(_JB):
                        sl = pl.ds((jb * _JB + k) * _L, _L)
                        v = xb[b][r, sl]
                        ob[b][r, sl] = (v * rs - ms) * gs[k] + bs[k]
                    return rc

                lax.fori_loop(0, _C, pass2, 0)

    issue_ins(0, 0)
    issue_ins(1, 1)

    def outer(g, carry):
        for slot in range(2):
            t = g * 2 + slot
            wait_ins(slot, t)

            @pl.when(t >= 2)
            def _():
                drain_outs(slot, t - 2)

            compute(slot)
            start_outs(slot, t)

            @pl.when(t < _NCHUNK - 2)
            def _():
                issue_ins(slot, t + 2)

        return carry

    lax.fori_loop(0, _NCHUNK // 2, outer, 0)
    drain_outs(0, _NCHUNK - 2)
    drain_outs(1, _NCHUNK - 1)


def _sc_part(mem, emb_table, gamma, beta, b_lo, b_hi):
    nb = b_hi - b_lo
    mesh = plsc.VectorSubcoreMesh(core_axis_name="c", subcore_axis_name="s")
    chunk_t = pltpu.VMEM((_C, HIDDEN), jnp.float32)
    run = pl.kernel(
        functools.partial(_sc_body, nb, b_lo),
        mesh=mesh,
        out_type=jax.ShapeDtypeStruct((nb, MEM_LENGTH, HIDDEN), jnp.float32),
        scratch_types=(
            [pltpu.VMEM((HIDDEN,), jnp.float32)] * 2
            + [chunk_t] * 2
            + [chunk_t] * (2 * nb)
            + [chunk_t] * (2 * nb)
            + [pltpu.VMEM((nb, _C, _L), jnp.float32)] * 2
            + [pltpu.SemaphoreType.DMA] * 4
        ),
    )
    return run(mem, emb_table, gamma, beta)


@jax.jit
def kernel(mem, emb_table, gamma, beta):
    tc_out = _tc_part(mem, emb_table, gamma, beta, _TCB, BATCH)
    sc_out = _sc_part(mem, emb_table, gamma, beta, _TCB, BATCH)
    tc_out, sc_out = lax.optimization_barrier((tc_out, sc_out))
    return lax.dynamic_update_slice(tc_out, sc_out, (_TCB, 0, 0))


# speedup vs baseline: 1.5002x; 1.0401x over previous
import functools

import jax
import jax.numpy as jnp
from jax import lax
from jax.experimental import pallas as pl
from jax.experimental.pallas import tpu as pltpu
from jax.experimental.pallas import tpu_sc as plsc

MEM_LENGTH = 8192
HIDDEN = 768
BATCH = 4

_TCB = 3
_BL = 512

_NC = 2
_NS = 16
_L = 16
_NW = _NC * _NS
_LPW = MEM_LENGTH // _NW
_C = 8
_NCHUNK = _LPW // _C
_NJ = HIDDEN // _L
_JB = 8

_GATHER_DNUMS = lax.GatherDimensionNumbers(
    offset_dims=(), collapsed_slice_dims=(0,), start_index_map=(0,))


def _lane_shuffle(v, perm):
    return lax.gather(v, perm[:, None], _GATHER_DNUMS, slice_sizes=(1,),
                      mode=lax.GatherScatterMode.PROMISE_IN_BOUNDS)


def _allsum_vec(v):
    idx = lax.iota(jnp.int32, 16)
    for sh in (8, 4, 2, 1):
        perm = lax.bitwise_xor(idx, jnp.int32(sh))
        v = v + _lane_shuffle(v, perm)
    return v


def _rsqrt_vec(x):
    i = lax.bitcast_convert_type(x, jnp.int32)
    i = jnp.int32(0x5F3759DF) - lax.shift_right_logical(i, 1)
    y = lax.bitcast_convert_type(i, jnp.float32)
    for _ in range(3):
        y = y * (1.5 - 0.5 * x * y * y)
    return y


def _ln_body(mem_ref, emb_ref, gamma_ref, beta_ref, out_ref):
    x = mem_ref[0] + emb_ref[...]
    mean = jnp.mean(x, axis=-1, keepdims=True)
    xc = x - mean
    var = jnp.mean(xc * xc, axis=-1, keepdims=True)
    inv = jax.lax.rsqrt(var + 1e-5)
    out_ref[0] = xc * inv * gamma_ref[...] + beta_ref[...]


_NBLK = MEM_LENGTH // _BL


def _tc_part(mem, emb_table, gamma, beta, n_blocks, out_nb):
    return pl.pallas_call(
        _ln_body,
        grid=(n_blocks,),
        in_specs=[
            pl.BlockSpec((1, _BL, HIDDEN), lambda k: (k // _NBLK, k % _NBLK, 0)),
            pl.BlockSpec((_BL, HIDDEN), lambda k: (k % _NBLK, 0)),
            pl.BlockSpec((HIDDEN,), lambda k: (0,)),
            pl.BlockSpec((HIDDEN,), lambda k: (0,)),
        ],
        out_specs=pl.BlockSpec((1, _BL, HIDDEN), lambda k: (k // _NBLK, k % _NBLK, 0)),
        out_shape=jax.ShapeDtypeStruct((out_nb, MEM_LENGTH, HIDDEN), jnp.float32),
    )(mem, emb_table, gamma, beta)


def _sc_body(nb, b_lo, l_base, lpw, nchunk, mem, emb, gamma, beta, out, *scr):
    cid = lax.axis_index("c")
    sid = lax.axis_index("s")
    wid = sid * _NC + cid
    base_g = l_base + wid * lpw
    base_l = wid * lpw

    gv, bv = scr[0], scr[1]
    es = scr[2:4]
    xs = (scr[4:4 + nb], scr[4 + nb:4 + 2 * nb])
    o0 = 4 + 2 * nb
    os_ = (scr[o0:o0 + nb], scr[o0 + nb:o0 + 2 * nb])
    rs_v, ms_v = scr[o0 + 2 * nb], scr[o0 + 2 * nb + 1]
    sins = scr[o0 + 2 * nb + 2:o0 + 2 * nb + 4]
    souts = scr[o0 + 2 * nb + 4:o0 + 2 * nb + 6]

    pltpu.sync_copy(gamma, gv)
    pltpu.sync_copy(beta, bv)
    zero = jnp.zeros((_L,), jnp.float32)

    def issue_ins(slot, t):
        l0 = base_g + t * _C
        pltpu.async_copy(emb.at[pl.ds(l0, _C)], es[slot], sins[slot])
        for b in range(nb):
            pltpu.async_copy(mem.at[b_lo + b, pl.ds(l0, _C)], xs[slot][b],
                             sins[slot])

    def wait_ins(slot, t):
        l0 = base_g + t * _C
        pltpu.make_async_copy(emb.at[pl.ds(l0, _C)], es[slot], sins[slot]).wait()
        for b in range(nb):
            pltpu.make_async_copy(
                mem.at[b_lo + b, pl.ds(l0, _C)], xs[slot][b], sins[slot]).wait()

    def start_outs(slot, t):
        l0 = base_l + t * _C
        for b in range(nb):
            pltpu.async_copy(os_[slot][b], out.at[b, pl.ds(l0, _C)], souts[slot])

    def drain_outs(slot, t):
        l0 = base_l + t * _C
        for b in range(nb):
            pltpu.make_async_copy(
                os_[slot][b], out.at[b, pl.ds(l0, _C)], souts[slot]).wait()

    def compute(slot):
        xb = xs[slot]
        ob = os_[slot]
        ev = es[slot]

        def pass1(r, rc):
            s = [[zero, zero] for _ in range(nb)]
            sq = [[zero, zero] for _ in range(nb)]
            for j in range(_NJ):
                sl = pl.ds(j * _L, _L)
                e = ev[r, sl]
                p = j & 1
                for b in range(nb):
                    v = xb[b][r, sl] + e
                    xb[b][r, sl] = v
                    s[b][p] = s[b][p] + v
                    sq[b][p] = sq[b][p] + v * v
            for b in range(nb):
                tot = _allsum_vec(s[b][0] + s[b][1])
                totsq = _allsum_vec(sq[b][0] + sq[b][1])
                mean = tot * (1.0 / HIDDEN)
                var = totsq * (1.0 / HIDDEN) - mean * mean
                rstd = _rsqrt_vec(var + 1e-5)
                rs_v[b, r, :] = rstd
                ms_v[b, r, :] = mean * rstd
            return rc

        lax.fori_loop(0, _C, pass1, 0)

        for jb in range(_NJ // _JB):
            gs = [gv[pl.ds((jb * _JB + k) * _L, _L)] for k in range(_JB)]
            bs = [bv[pl.ds((jb * _JB + k) * _L, _L)] for k in range(_JB)]
            for b in range(nb):
                def pass2(r, rc, jb=jb, b=b, gs=gs, bs=bs):
                    rs = rs_v[b, r, :]
                    ms = ms_v[b, r, :]
                    for k in range(_JB):
                        sl = pl.ds((jb * _JB + k) * _L, _L)
                        v = xb[b][r, sl]
                        ob[b][r, sl] = (v * rs - ms) * gs[k] + bs[k]
                    return rc

                lax.fori_loop(0, _C, pass2, 0)

    issue_ins(0, 0)
    issue_ins(1, 1)

    def outer(g, carry):
        for slot in range(2):
            t = g * 2 + slot
            wait_ins(slot, t)

            @pl.when(t >= 2)
            def _():
                drain_outs(slot, t - 2)

            compute(slot)
            start_outs(slot, t)

            @pl.when(t < nchunk - 2)
            def _():
                issue_ins(slot, t + 2)

        return carry

    lax.fori_loop(0, nchunk // 2, outer, 0)
    drain_outs(0, nchunk - 2)
    drain_outs(1, nchunk - 1)


def _sc_part(mem, emb_table, gamma, beta, b_lo, b_hi, l_base, l_span):
    nb = b_hi - b_lo
    lpw = l_span // _NW
    nchunk = lpw // _C
    mesh = plsc.VectorSubcoreMesh(core_axis_name="c", subcore_axis_name="s")
    chunk_t = pltpu.VMEM((_C, HIDDEN), jnp.float32)
    run = pl.kernel(
        functools.partial(_sc_body, nb, b_lo, l_base, lpw, nchunk),
        mesh=mesh,
        out_type=jax.ShapeDtypeStruct((nb, l_span, HIDDEN), jnp.float32),
        scratch_types=(
            [pltpu.VMEM((HIDDEN,), jnp.float32)] * 2
            + [chunk_t] * 2
            + [chunk_t] * (2 * nb)
            + [chunk_t] * (2 * nb)
            + [pltpu.VMEM((nb, _C, _L), jnp.float32)] * 2
            + [pltpu.SemaphoreType.DMA] * 4
        ),
    )
    return run(mem, emb_table, gamma, beta)


_SC_L0 = 4096
_SC_SPAN = MEM_LENGTH - _SC_L0


@jax.jit
def kernel(mem, emb_table, gamma, beta):
    n_blocks = _TCB * _NBLK + _SC_L0 // _BL
    tc_out = _tc_part(mem, emb_table, gamma, beta, n_blocks, BATCH)
    sc_out = _sc_part(mem, emb_table, gamma, beta, _TCB, BATCH, _SC_L0, _SC_SPAN)
    tc_out, sc_out = lax.optimization_barrier((tc_out, sc_out))
    return lax.dynamic_update_slice(tc_out, sc_out, (_TCB, _SC_L0, 0))
